# Initial kernel scaffold; baseline (speedup 1.0000x reference)
import jax
import jax.numpy as jnp
from jax.experimental import pallas as pl


def kernel(F_v, F_i, W1, b1, W2, b2, temperature):
    raise NotImplementedError("write your pallas kernel here")


# speedup vs baseline: 20.3598x; 20.3598x over previous
import functools

import jax
import jax.numpy as jnp
import numpy as np
from jax.experimental import pallas as pl
from jax.experimental.pallas import tpu as pltpu

B, C, H, W = 4, 512, 32, 32
N = H * W
K = 16
RT = 128
NT = N // RT

_NEG_INF = np.float32(-np.inf)


def _mlp_mask(Cmat, W1, b1, W2, b2_scalar):
    h = jax.lax.dot_general(Cmat, W1, (((1,), (1,)), ((), ())),
                            preferred_element_type=jnp.float32)
    h = jnp.maximum(h + b1, 0.0)
    z = jnp.sum(h * W2, axis=1, keepdims=True) + b2_scalar
    return 1.0 / (1.0 + jnp.exp(-z))


def _main_kernel(xv_ref, xi_ref, w1_ref, b1_ref, w2_ref, b2_ref, t_ref,
                 ov_ref, oi_ref, loss_ref):
    b = pl.program_id(0)
    t = pl.program_id(1)

    xv = xv_ref[0]
    xi = xi_ref[0]
    inv_t = 1.0 / t_ref[0]

    nv = xv * (1.0 / (jnp.sqrt(jnp.sum(xv * xv, axis=1, keepdims=True)) + 1e-6))
    ni = xi * (1.0 / (jnp.sqrt(jnp.sum(xi * xi, axis=1, keepdims=True)) + 1e-6))

    raw_v = xv_ref[0, pl.ds(t * RT, RT), :]
    raw_i = xi_ref[0, pl.ds(t * RT, RT), :]
    rows_v = raw_v * (1.0 / (jnp.sqrt(jnp.sum(raw_v * raw_v, axis=1,
                                              keepdims=True)) + 1e-6))
    rows_i = raw_i * (1.0 / (jnp.sqrt(jnp.sum(raw_i * raw_i, axis=1,
                                              keepdims=True)) + 1e-6))

    dn = (((1,), (1,)), ((), ()))
    sim_v = jax.lax.dot_general(rows_v, nv, dn,
                                preferred_element_type=jnp.float32)
    sim_i = jax.lax.dot_general(rows_i, ni, dn,
                                preferred_element_type=jnp.float32)

    col = jax.lax.broadcasted_iota(jnp.int32, (RT, N), 1).astype(jnp.float32)
    row_g = (t * RT
             + jax.lax.broadcasted_iota(jnp.int32, (RT, N), 0)
             ).astype(jnp.float32)
    neg = jnp.where(col == row_g, _NEG_INF, sim_v - 1.0)

    cvs = []
    cis = []
    for _ in range(K):
        m = jnp.max(neg, axis=1, keepdims=True)
        cand = jnp.where(neg == m, col, np.float32(N))
        am = jnp.min(cand, axis=1, keepdims=True)
        onehot = col == am
        cvs.append(jnp.sum(jnp.where(onehot, sim_v, 0.0), axis=1, keepdims=True))
        cis.append(jnp.sum(jnp.where(onehot, sim_i, 0.0), axis=1, keepdims=True))
        neg = jnp.where(onehot, _NEG_INF, neg)

    Cv = jnp.concatenate(cvs, axis=1) * inv_t
    Ci = jnp.concatenate(cis, axis=1) * inv_t

    b2s = b2_ref[0]
    mv = _mlp_mask(Cv, w1_ref[...], b1_ref[...], w2_ref[...], b2s)
    mi = _mlp_mask(Ci, w1_ref[...], b1_ref[...], w2_ref[...], b2s)

    ov_ref[0] = raw_v * mv
    oi_ref[0] = raw_i * mi

    eps = np.float32(1e-12)
    cvn = Cv / jnp.maximum(jnp.sqrt(jnp.sum(Cv * Cv, axis=1, keepdims=True)), eps)
    cin = Ci / jnp.maximum(jnp.sqrt(jnp.sum(Ci * Ci, axis=1, keepdims=True)), eps)
    part = jnp.sum((cvn - cin) ** 2).reshape(1, 1)

    @pl.when((b == 0) & (t == 0))
    def _():
        loss_ref[:, :] = jnp.zeros((1, 1), jnp.float32)

    loss_ref[:, :] += part


@jax.jit
def kernel(F_v, F_i, W1, b1, W2, b2, temperature):
    b, c, h, w = F_v.shape
    Fv = F_v.reshape(b, c, h * w).transpose(0, 2, 1)
    Fi = F_i.reshape(b, c, h * w).transpose(0, 2, 1)

    grid = (B, NT)
    full = pl.BlockSpec((1, N, C), lambda bb, tt: (bb, 0, 0))
    outb = pl.BlockSpec((1, RT, C), lambda bb, tt: (bb, tt, 0))
    rep = lambda shape: pl.BlockSpec(shape, lambda bb, tt: tuple(0 for _ in shape))

    ov, oi, loss = pl.pallas_call(
        _main_kernel,
        grid=grid,
        in_specs=[
            full, full,
            rep((32, K)), rep((1, 32)), rep((1, 32)),
            pl.BlockSpec(memory_space=pltpu.SMEM),
            pl.BlockSpec(memory_space=pltpu.SMEM),
        ],
        out_specs=[
            outb, outb,
            pl.BlockSpec((1, 1), lambda bb, tt: (0, 0)),
        ],
        out_shape=[
            jax.ShapeDtypeStruct((B, N, C), jnp.float32),
            jax.ShapeDtypeStruct((B, N, C), jnp.float32),
            jax.ShapeDtypeStruct((1, 1), jnp.float32),
        ],
    )(Fv, Fi, W1, b1.reshape(1, 32), W2, b2.reshape(1),
      temperature.reshape(1))

    F_v_den = ov.transpose(0, 2, 1).reshape(b, c, h, w)
    F_i_den = oi.transpose(0, 2, 1).reshape(b, c, h, w)
    loss_syn = (loss[0, 0] * (100.0 / (B * N * K))).astype(jnp.float32)
    return (F_v_den, F_i_den, loss_syn)
